# baseline (speedup 1.0000x reference)
import functools

import jax
import jax.numpy as jnp
from jax import lax
from jax.experimental import pallas as pl
from jax.experimental.pallas import tpu as pltpu
from jax.experimental.pallas import tpu_sc as plsc

VOCAB = 1000000
HIDDEN = 64
OUT = 1000
B = 20480

NC = 2
NS = 16
NW = NC * NS
B_PER_W = B // NW
CHUNK = 128
N_CHUNK = B_PER_W // CHUNK


def _sc_gather(idx, emb):
    mesh = plsc.VectorSubcoreMesh(core_axis_name="c", subcore_axis_name="s")

    @functools.partial(
        pl.kernel,
        mesh=mesh,
        out_type=jax.ShapeDtypeStruct((B, HIDDEN), jnp.float32),
        scratch_types=[
            pltpu.VMEM((N_CHUNK, CHUNK), jnp.int32),
            pltpu.VMEM((B_PER_W, HIDDEN), jnp.float32),
            pltpu.SemaphoreType.DMA,
        ],
        compiler_params=pltpu.CompilerParams(use_tc_tiling_on_sc=False),
    )
    def gather_kernel(idx_hbm, emb_hbm, out_hbm, idx_v, rows_v, sem):
        wid = lax.axis_index("s") * NC + lax.axis_index("c")
        base = wid * B_PER_W
        pltpu.sync_copy(idx_hbm.at[wid], idx_v)
        copies = []
        for j in range(N_CHUNK):
            copies.append(
                pltpu.async_copy(
                    emb_hbm.at[idx_v.at[j]],
                    rows_v.at[pl.ds(j * CHUNK, CHUNK)],
                    sem,
                )
            )
        for c in copies:
            c.wait()
        pltpu.sync_copy(rows_v, out_hbm.at[pl.ds(base, B_PER_W)])

    return gather_kernel(idx.reshape(NW, N_CHUNK, CHUNK), emb)


def _tc_decode(x, wt, b2):
    bm = 1024
    grid = (B // bm,)

    def body(x_ref, w_ref, b_ref, o_ref):
        xr = jnp.maximum(x_ref[...], 0.0)
        o_ref[...] = (
            jnp.dot(xr, w_ref[...], preferred_element_type=jnp.float32)
            + b_ref[...]
        )

    return pl.pallas_call(
        body,
        grid=grid,
        in_specs=[
            pl.BlockSpec((bm, HIDDEN), lambda i: (i, 0)),
            pl.BlockSpec((HIDDEN, OUT), lambda i: (0, 0)),
            pl.BlockSpec((1, OUT), lambda i: (0, 0)),
        ],
        out_specs=pl.BlockSpec((bm, OUT), lambda i: (i, 0)),
        out_shape=jax.ShapeDtypeStruct((B, OUT), jnp.float32),
    )(x, wt, b2)


def kernel(inp, hidden, emb, W, b):
    idx = inp.reshape(-1).astype(jnp.int32)
    x = _sc_gather(idx, emb)
    logits = _tc_decode(x, W.T, b.reshape(1, OUT))
    return (logits, hidden)


# speedup vs baseline: 1.0014x; 1.0014x over previous
import functools

import jax
import jax.numpy as jnp
from jax import lax
from jax.experimental import pallas as pl
from jax.experimental.pallas import tpu as pltpu
from jax.experimental.pallas import tpu_sc as plsc

VOCAB = 1000000
HIDDEN = 64
OUT = 1000
B = 20480
DPAIR = 2 * HIDDEN

NC = 2
NS = 16
NW = NC * NS
B_PER_W = B // NW
CHUNK = 128
N_CHUNK = B_PER_W // CHUNK


def _sc_gather_pairs(idx_pair, emb2):
    mesh = plsc.VectorSubcoreMesh(core_axis_name="c", subcore_axis_name="s")

    @functools.partial(
        pl.kernel,
        mesh=mesh,
        out_type=jax.ShapeDtypeStruct((B, DPAIR), jnp.float32),
        scratch_types=[
            pltpu.VMEM((N_CHUNK, CHUNK), jnp.int32),
            pltpu.VMEM((B_PER_W, DPAIR), jnp.float32),
            pltpu.SemaphoreType.DMA,
        ],
    )
    def gather_kernel(idx_hbm, emb_hbm, out_hbm, idx_v, rows_v, sem):
        wid = lax.axis_index("s") * NC + lax.axis_index("c")
        base = wid * B_PER_W
        pltpu.sync_copy(idx_hbm.at[wid], idx_v)
        copies = []
        for j in range(N_CHUNK):
            copies.append(
                pltpu.async_copy(
                    emb_hbm.at[idx_v.at[j]],
                    rows_v.at[pl.ds(j * CHUNK, CHUNK)],
                    sem,
                )
            )
        for c in copies:
            c.wait()
        pltpu.sync_copy(rows_v, out_hbm.at[pl.ds(base, B_PER_W)])

    return gather_kernel(idx_pair.reshape(NW, N_CHUNK, CHUNK), emb2)


def _tc_decode(x, par, wt, b2):
    bm = 1024
    grid = (B // bm,)

    def body(x_ref, p_ref, w_ref, b_ref, o_ref):
        lo = x_ref[:, :HIDDEN]
        hi = x_ref[:, HIDDEN:]
        sel = jnp.where(p_ref[...] > 0, hi, lo)
        xr = jnp.maximum(sel, 0.0)
        o_ref[...] = (
            jnp.dot(xr, w_ref[...], preferred_element_type=jnp.float32)
            + b_ref[...]
        )

    return pl.pallas_call(
        body,
        grid=grid,
        in_specs=[
            pl.BlockSpec((bm, DPAIR), lambda i: (i, 0)),
            pl.BlockSpec((bm, 1), lambda i: (i, 0)),
            pl.BlockSpec((HIDDEN, OUT), lambda i: (0, 0)),
            pl.BlockSpec((1, OUT), lambda i: (0, 0)),
        ],
        out_specs=pl.BlockSpec((bm, OUT), lambda i: (i, 0)),
        out_shape=jax.ShapeDtypeStruct((B, OUT), jnp.float32),
    )(x, par, wt, b2)


def kernel(inp, hidden, emb, W, b):
    idx = inp.reshape(-1).astype(jnp.int32)
    emb2 = emb.reshape(VOCAB // 2, DPAIR)
    idx_pair = idx // 2
    parity = (idx & 1).reshape(B, 1)
    x = _sc_gather_pairs(idx_pair, emb2)
    logits = _tc_decode(x, parity, W.T, b.reshape(1, OUT))
    return (logits, hidden)


# speedup vs baseline: 1.5657x; 1.5634x over previous
import functools

import jax
import jax.numpy as jnp
from jax import lax
from jax.experimental import pallas as pl
from jax.experimental.pallas import tpu as pltpu
from jax.experimental.pallas import tpu_sc as plsc

VOCAB = 1000000
HIDDEN = 64
OUT = 1000
B = 20480

NC = 2
NS = 16
NW = NC * NS
B_PER_W = B // NW


def _sc_gather(idx, emb):
    mesh = plsc.VectorSubcoreMesh(core_axis_name="c", subcore_axis_name="s")

    @functools.partial(
        pl.kernel,
        mesh=mesh,
        out_type=jax.ShapeDtypeStruct((B, HIDDEN), jnp.float32),
        scratch_types=[
            pltpu.VMEM((B_PER_W,), jnp.int32),
            pltpu.VMEM((B_PER_W, HIDDEN), jnp.float32),
            pltpu.SemaphoreType.DMA,
        ],
    )
    def gather_kernel(idx_hbm, emb_hbm, out_hbm, idx_s, rows_v, sem):
        wid = lax.axis_index("s") * NC + lax.axis_index("c")
        base = wid * B_PER_W
        pltpu.sync_copy(idx_hbm.at[pl.ds(base, B_PER_W)], idx_s)

        def fire(g, carry):
            vec = idx_s[pl.ds(g * 16, 16)]
            for lane in range(16):
                pltpu.async_copy(
                    emb_hbm.at[pl.ds(vec[lane], 1)],
                    rows_v.at[pl.ds(g * 16 + lane, 1)],
                    sem,
                )
            return carry

        lax.fori_loop(0, B_PER_W // 16, fire, 0)
        pltpu.make_async_copy(
            emb_hbm.at[pl.ds(0, B_PER_W)], rows_v, sem
        ).wait()
        pltpu.sync_copy(rows_v, out_hbm.at[pl.ds(base, B_PER_W)])

    return gather_kernel(idx, emb)


def _tc_decode(x, wt, b2):
    bm = 1024
    grid = (B // bm,)

    def body(x_ref, w_ref, b_ref, o_ref):
        xr = jnp.maximum(x_ref[...], 0.0)
        o_ref[...] = (
            jnp.dot(xr, w_ref[...], preferred_element_type=jnp.float32)
            + b_ref[...]
        )

    return pl.pallas_call(
        body,
        grid=grid,
        in_specs=[
            pl.BlockSpec((bm, HIDDEN), lambda i: (i, 0)),
            pl.BlockSpec((HIDDEN, OUT), lambda i: (0, 0)),
            pl.BlockSpec((1, OUT), lambda i: (0, 0)),
        ],
        out_specs=pl.BlockSpec((bm, OUT), lambda i: (i, 0)),
        out_shape=jax.ShapeDtypeStruct((B, OUT), jnp.float32),
    )(x, wt, b2)


def kernel(inp, hidden, emb, W, b):
    idx = inp.reshape(-1).astype(jnp.int32)
    x = _sc_gather(idx, emb)
    logits = _tc_decode(x, W.T, b.reshape(1, OUT))
    return (logits, hidden)


# speedup vs baseline: 2.2503x; 1.4373x over previous
import functools

import jax
import jax.numpy as jnp
from jax import lax
from jax.experimental import pallas as pl
from jax.experimental.pallas import tpu as pltpu
from jax.experimental.pallas import tpu_sc as plsc

VOCAB = 1000000
HIDDEN = 64
OUT = 1000
B = 20480

NC = 2
NS = 16
NW = NC * NS
B_PER_W = B // NW


def _tc_transpose(emb_t):
    bk = 8192
    grid = (pl.cdiv(VOCAB, bk),)

    def body(x_ref, o_ref):
        o_ref[...] = x_ref[...].T

    return pl.pallas_call(
        body,
        grid=grid,
        in_specs=[pl.BlockSpec((HIDDEN, bk), lambda i: (0, i))],
        out_specs=pl.BlockSpec((bk, HIDDEN), lambda i: (i, 0)),
        out_shape=jax.ShapeDtypeStruct((VOCAB, HIDDEN), jnp.float32),
    )(emb_t)


def _sc_gather(idx, emb_rows):
    mesh = plsc.VectorSubcoreMesh(core_axis_name="c", subcore_axis_name="s")

    @functools.partial(
        pl.kernel,
        mesh=mesh,
        out_type=jax.ShapeDtypeStruct((B, HIDDEN), jnp.float32),
        scratch_types=[
            pltpu.VMEM((B_PER_W,), jnp.int32),
            pltpu.VMEM((B_PER_W, HIDDEN), jnp.float32),
            pltpu.SemaphoreType.DMA,
        ],
    )
    def gather_kernel(idx_hbm, emb_hbm, out_hbm, idx_v, rows_v, sem):
        wid = lax.axis_index("s") * NC + lax.axis_index("c")
        base = wid * B_PER_W
        pltpu.sync_copy(idx_hbm.at[pl.ds(base, B_PER_W)], idx_v)

        def fire(g, carry):
            vec = idx_v[pl.ds(g * 16, 16)]
            for lane in range(16):
                pltpu.async_copy(
                    emb_hbm.at[pl.ds(vec[lane], 1)],
                    rows_v.at[pl.ds(g * 16 + lane, 1)],
                    sem,
                )
            return carry

        lax.fori_loop(0, B_PER_W // 16, fire, 0)
        pltpu.make_async_copy(
            emb_hbm.at[pl.ds(0, B_PER_W)], rows_v, sem
        ).wait()
        pltpu.sync_copy(rows_v, out_hbm.at[pl.ds(base, B_PER_W)])

    return gather_kernel(idx, emb_rows)


def _tc_decode_t(x_t, w, b2):
    bm = 2048
    grid = (B // bm,)

    def body(w_ref, x_ref, b_ref, o_ref):
        xr = jnp.maximum(x_ref[...], 0.0)
        o_ref[...] = (
            jnp.dot(w_ref[...], xr, preferred_element_type=jnp.float32)
            + b_ref[...]
        )

    return pl.pallas_call(
        body,
        grid=grid,
        in_specs=[
            pl.BlockSpec((OUT, HIDDEN), lambda i: (0, 0)),
            pl.BlockSpec((HIDDEN, bm), lambda i: (0, i)),
            pl.BlockSpec((OUT, 1), lambda i: (0, 0)),
        ],
        out_specs=pl.BlockSpec((OUT, bm), lambda i: (0, i)),
        out_shape=jax.ShapeDtypeStruct((OUT, B), jnp.float32),
    )(w, x_t, b2)


def kernel(inp, hidden, emb, W, b):
    idx = inp.reshape(-1).astype(jnp.int32)
    emb_rows = _tc_transpose(emb.T)
    x = _sc_gather(idx, emb_rows)
    logits_t = _tc_decode_t(x.T, W, b.reshape(OUT, 1))
    return (logits_t.T, hidden)


# speedup vs baseline: 2.8190x; 1.2527x over previous
import functools

import jax
import jax.numpy as jnp
from jax import lax
from jax.experimental import pallas as pl
from jax.experimental.pallas import tpu as pltpu
from jax.experimental.pallas import tpu_sc as plsc

VOCAB = 1000000
HIDDEN = 64
OUT = 1000
B = 20480

NC = 2
NS = 16
NW = NC * NS

CH = 256
SPAN = 31232
NCH_LO = SPAN // CH
SPAN_HI = VOCAB - 64 - 31 * SPAN
NCH_HI = SPAN_HI // CH
TAIL0 = VOCAB - 64
RING = 64


def _sc_gather_stream(idx, emb_t, tail_t):
    mesh = plsc.VectorSubcoreMesh(core_axis_name="c", subcore_axis_name="s")

    @functools.partial(
        pl.kernel,
        mesh=mesh,
        out_type=jax.ShapeDtypeStruct((B, HIDDEN), jnp.float32),
        scratch_types=[
            pltpu.VMEM((B,), jnp.int32),
            pltpu.VMEM((B,), jnp.int32),
            pltpu.VMEM((B,), jnp.int32),
            pltpu.VMEM((2, HIDDEN, CH), jnp.float32),
            pltpu.VMEM((RING, HIDDEN), jnp.float32),
            pltpu.VMEM((HIDDEN, 64), jnp.float32),
            pltpu.SMEM((4,), jnp.int32),
            pltpu.SemaphoreType.DMA,
            pltpu.SemaphoreType.DMA,
        ],
        compiler_params=pltpu.CompilerParams(needs_layout_passes=False),
    )
    def gather_kernel(idx_hbm, emb_hbm, tail_hbm, out_hbm,
                      idx_v, pos_v, cpos_v, chunk_v, ring_v, tail_v,
                      cnt_s, csem, rsem):
        wid = lax.axis_index("s") * NC + lax.axis_index("c")
        last = wid == NW - 1
        base = wid * SPAN
        himark = jnp.where(last, VOCAB, base + SPAN)
        nch = jnp.where(last, NCH_HI, NCH_LO)
        lane16 = jax.lax.iota(jnp.int32, 16)

        pltpu.sync_copy(idx_hbm, idx_v)
        cnt_s[0] = 0
        cnt_s[1] = 0

        def build(g, n):
            iv = idx_v[pl.ds(g * 16, 16)]
            mm = (iv >= base) & (iv < himark)
            inc = plsc.cumsum(jnp.where(mm, 1, 0))
            plsc.store_scatter(pos_v, [n + inc - 1], g * 16 + lane16, mask=mm)
            return n + inc[15]

        n_mine = lax.fori_loop(0, B // 16, build, 0)
        ngrp = (n_mine + 15) // 16

        def extract_hits(chunk_ref, clo, m_hits):
            def one_group(g2, _):
                valid = g2 * 16 + lane16 < m_hits
                cp = jnp.where(valid, cpos_v[pl.ds(g2 * 16, 16)], 0)
                iv = plsc.load_gather(idx_v, [cp])
                cols = jnp.where(valid, iv - clo, 0)
                valid_i = jnp.where(valid, 1, 0)
                for lane in range(16):
                    @pl.when(valid_i[lane] == 1)
                    def _():
                        slot = lax.rem(cnt_s[0], RING)
                        col16 = jnp.full((16,), cols[lane], jnp.int32)
                        for j in range(HIDDEN // 16):
                            vals = plsc.load_gather(
                                chunk_ref, [lane16 + j * 16, col16]
                            )
                            ring_v[slot, pl.ds(j * 16, 16)] = vals
                        pltpu.async_copy(
                            ring_v.at[pl.ds(slot, 1)],
                            out_hbm.at[pl.ds(cp[lane], 1)],
                            rsem,
                        )
                        cnt_s[0] = cnt_s[0] + 1
                        cnt_s[1] = cnt_s[1] + 1

                @pl.when(cnt_s[1] >= RING - 16)
                def _():
                    def drain(_, c):
                        pltpu.make_async_copy(
                            out_hbm.at[pl.ds(0, 1)],
                            ring_v.at[pl.ds(0, 1)],
                            rsem,
                        ).wait()
                        return c
                    lax.fori_loop(0, cnt_s[1], drain, 0)
                    cnt_s[1] = 0
                return 0

            lax.fori_loop(0, (m_hits + 15) // 16, one_group, 0)

        def scan_hits(clo, chi):
            def ga(g, m):
                valid = g * 16 + lane16 < n_mine
                pv = jnp.where(valid, pos_v[pl.ds(g * 16, 16)], 0)
                iv = plsc.load_gather(idx_v, [pv])
                mm = valid & (iv >= clo) & (iv < chi)
                inc = plsc.cumsum(jnp.where(mm, 1, 0))
                plsc.store_scatter(cpos_v, [m + inc - 1], pv, mask=mm)
                return m + inc[15]

            return lax.fori_loop(0, ngrp, ga, 0)

        pltpu.async_copy(
            emb_hbm.at[:, pl.ds(pl.multiple_of(base, 128), CH)],
            chunk_v.at[0], csem,
        )

        def chunk_step(c, _):
            @pl.when(c + 1 < nch)
            def _():
                off = base + (c + 1) * CH
                pltpu.async_copy(
                    emb_hbm.at[:, pl.ds(pl.multiple_of(off, 128), CH)],
                    chunk_v.at[lax.rem(c + 1, 2)], csem,
                )
            pltpu.make_async_copy(
                emb_hbm.at[:, pl.ds(0, CH)], chunk_v.at[0], csem
            ).wait()
            clo = base + c * CH
            m_hits = scan_hits(clo, clo + CH)
            extract_hits(chunk_v.at[lax.rem(c, 2)], clo, m_hits)
            return 0

        lax.fori_loop(0, nch, chunk_step, 0)

        @pl.when(last)
        def _():
            pltpu.sync_copy(tail_hbm, tail_v)
            m_hits = scan_hits(TAIL0, VOCAB)
            extract_hits(tail_v, TAIL0, m_hits)

        def drain(_, c):
            pltpu.make_async_copy(
                out_hbm.at[pl.ds(0, 1)], ring_v.at[pl.ds(0, 1)], rsem
            ).wait()
            return c
        lax.fori_loop(0, cnt_s[1], drain, 0)

    return gather_kernel(idx, emb_t, tail_t)


def _tc_decode_t(x_t, w, b2):
    bm = 2048
    grid = (B // bm,)

    def body(w_ref, x_ref, b_ref, o_ref):
        xr = jnp.maximum(x_ref[...], 0.0)
        o_ref[...] = (
            jnp.dot(w_ref[...], xr, preferred_element_type=jnp.float32)
            + b_ref[...]
        )

    return pl.pallas_call(
        body,
        grid=grid,
        in_specs=[
            pl.BlockSpec((OUT, HIDDEN), lambda i: (0, 0)),
            pl.BlockSpec((HIDDEN, bm), lambda i: (0, i)),
            pl.BlockSpec((OUT, 1), lambda i: (0, 0)),
        ],
        out_specs=pl.BlockSpec((OUT, bm), lambda i: (0, i)),
        out_shape=jax.ShapeDtypeStruct((OUT, B), jnp.float32),
    )(w, x_t, b2)


def kernel(inp, hidden, emb, W, b):
    idx = inp.reshape(-1).astype(jnp.int32)
    emb_t = emb.T
    tail_t = lax.slice(emb_t, (0, TAIL0), (HIDDEN, VOCAB))
    x = _sc_gather_stream(idx, emb_t, tail_t)
    logits_t = _tc_decode_t(x.T, W, b.reshape(OUT, 1))
    return (logits_t.T, hidden)
